# baseline (speedup 1.0000x reference)
import functools

import jax
import jax.numpy as jnp
from jax import lax
from jax.experimental import pallas as pl
from jax.experimental.pallas import tpu as pltpu
from jax.experimental.pallas import tpu_sc as plsc

POOL_SIZE = 65536
N_TOK = 16384
NOPE_DIM = 512
ROPE_DIM = 64
ROW_DIM = NOPE_DIM + ROPE_DIM

_NC, _NS = 2, 16
_NW = _NC * _NS
_B_PER_W = N_TOK // _NW
_CHUNK = 128
_N_CHUNKS = _B_PER_W // _CHUNK


def _gather_body(kv_hbm, loc_hbm, nope_hbm, rope_hbm, idx_v, rows_v, sem):
    wid = lax.axis_index("s") * _NC + lax.axis_index("c")
    base = wid * _B_PER_W
    pltpu.sync_copy(loc_hbm.at[pl.ds(base, _B_PER_W)], idx_v)
    for j in range(_N_CHUNKS):
        idx_chunk = idx_v.at[pl.ds(j * _CHUNK, _CHUNK)]
        pltpu.async_copy(kv_hbm.at[idx_chunk], rows_v, sem).wait()
        row0 = base + j * _CHUNK
        pltpu.sync_copy(rows_v.at[:, pl.ds(0, NOPE_DIM)],
                        nope_hbm.at[pl.ds(row0, _CHUNK)])
        pltpu.sync_copy(rows_v.at[:, pl.ds(NOPE_DIM, ROPE_DIM)],
                        rope_hbm.at[pl.ds(row0, _CHUNK)])


@jax.jit
def _mla_gather(kv2d, loc32):
    mesh = plsc.VectorSubcoreMesh(core_axis_name="c", subcore_axis_name="s")
    f = functools.partial(
        pl.kernel,
        mesh=mesh,
        out_type=(
            jax.ShapeDtypeStruct((N_TOK, NOPE_DIM), jnp.float32),
            jax.ShapeDtypeStruct((N_TOK, ROPE_DIM), jnp.float32),
        ),
        scratch_types=[
            pltpu.VMEM((_B_PER_W,), jnp.int32),
            pltpu.VMEM((_CHUNK, ROW_DIM), jnp.float32),
            pltpu.SemaphoreType.DMA,
        ],
        compiler_params=pltpu.CompilerParams(use_tc_tiling_on_sc=False),
    )(_gather_body)
    return f(kv2d, loc32)


def kernel(kv_buffer, loc, cache_k_nope, cache_k_rope):
    kv2d = kv_buffer.reshape(POOL_SIZE, ROW_DIM)
    loc32 = loc.astype(jnp.int32)
    nope, rope = _mla_gather(kv2d, loc32)
    return (nope.reshape(N_TOK, 1, NOPE_DIM).astype(cache_k_nope.dtype),
            rope.reshape(N_TOK, 1, ROPE_DIM).astype(cache_k_rope.dtype))


# speedup vs baseline: 1.0012x; 1.0012x over previous
import functools

import jax
import jax.numpy as jnp
from jax import lax
from jax.experimental import pallas as pl
from jax.experimental.pallas import tpu as pltpu
from jax.experimental.pallas import tpu_sc as plsc

POOL_SIZE = 65536
N_TOK = 16384
NOPE_DIM = 512
ROPE_DIM = 64
ROW_DIM = NOPE_DIM + ROPE_DIM

_NC, _NS = 2, 16
_NW = _NC * _NS
_B_PER_W = N_TOK // _NW
_CHUNK = 64
_N_CHUNKS = _B_PER_W // _CHUNK
_N_BUF = 3


def _gather_body(kv_hbm, loc_hbm, nope_hbm, rope_hbm,
                 idx_v, rows0, rows1, rows2, gsem, nsem, rsem):
    wid = lax.axis_index("s") * _NC + lax.axis_index("c")
    base = wid * _B_PER_W
    pltpu.sync_copy(loc_hbm.at[pl.ds(base, _B_PER_W)], idx_v)
    bufs = (rows0, rows1, rows2)

    def start_gather(j):
        idx_chunk = idx_v.at[pl.ds(j * _CHUNK, _CHUNK)]
        return pltpu.async_copy(kv_hbm.at[idx_chunk], bufs[j % _N_BUF], gsem)

    def start_writes(j):
        buf = bufs[j % _N_BUF]
        row0 = base + j * _CHUNK
        wn = pltpu.async_copy(buf.at[:, pl.ds(0, NOPE_DIM)],
                              nope_hbm.at[pl.ds(row0, _CHUNK)], nsem)
        wr = pltpu.async_copy(buf.at[:, pl.ds(NOPE_DIM, ROPE_DIM)],
                              rope_hbm.at[pl.ds(row0, _CHUNK)], rsem)
        return wn, wr

    g = {0: start_gather(0)}
    w = {}
    for j in range(_N_CHUNKS):
        if j + 1 < _N_CHUNKS:
            if j - 2 >= 0:
                for c in w.pop(j - 2):
                    c.wait()
            g[j + 1] = start_gather(j + 1)
        g.pop(j).wait()
        w[j] = start_writes(j)
    for j in sorted(w):
        for c in w.pop(j):
            c.wait()


@jax.jit
def _mla_gather(kv2d, loc32):
    mesh = plsc.VectorSubcoreMesh(core_axis_name="c", subcore_axis_name="s")
    f = functools.partial(
        pl.kernel,
        mesh=mesh,
        out_type=(
            jax.ShapeDtypeStruct((N_TOK, NOPE_DIM), jnp.float32),
            jax.ShapeDtypeStruct((N_TOK, ROPE_DIM), jnp.float32),
        ),
        scratch_types=[
            pltpu.VMEM((_B_PER_W,), jnp.int32),
            pltpu.VMEM((_CHUNK, ROW_DIM), jnp.float32),
            pltpu.VMEM((_CHUNK, ROW_DIM), jnp.float32),
            pltpu.VMEM((_CHUNK, ROW_DIM), jnp.float32),
            pltpu.SemaphoreType.DMA,
            pltpu.SemaphoreType.DMA,
            pltpu.SemaphoreType.DMA,
        ],
        compiler_params=pltpu.CompilerParams(use_tc_tiling_on_sc=False),
    )(_gather_body)
    return f(kv2d, loc32)


def kernel(kv_buffer, loc, cache_k_nope, cache_k_rope):
    kv2d = kv_buffer.reshape(POOL_SIZE, ROW_DIM)
    loc32 = loc.astype(jnp.int32)
    nope, rope = _mla_gather(kv2d, loc32)
    return (nope.reshape(N_TOK, 1, NOPE_DIM).astype(cache_k_nope.dtype),
            rope.reshape(N_TOK, 1, ROPE_DIM).astype(cache_k_rope.dtype))


# speedup vs baseline: 1.4810x; 1.4793x over previous
import functools

import jax
import jax.numpy as jnp
from jax import lax
from jax.experimental import pallas as pl
from jax.experimental.pallas import tpu as pltpu
from jax.experimental.pallas import tpu_sc as plsc

POOL_SIZE = 65536
N_TOK = 16384
NOPE_DIM = 512
ROPE_DIM = 64
ROW_DIM = NOPE_DIM + ROPE_DIM

_NC, _NS = 2, 16
_NW = _NC * _NS
_B_PER_W = N_TOK // _NW
_CHUNK = 64
_N_CHUNKS = _B_PER_W // _CHUNK


def _gather_body(kv_hbm, tail_hbm, loc_hbm, nope_hbm, rope_hbm,
                 idx_v, nope_v, tail_v, gsem):
    wid = lax.axis_index("s") * _NC + lax.axis_index("c")
    base = wid * _B_PER_W
    pltpu.sync_copy(loc_hbm.at[pl.ds(base, _B_PER_W)], idx_v)
    for j in range(_N_CHUNKS):
        idx_chunk = idx_v.at[pl.ds(j * _CHUNK, _CHUNK)]
        gn = pltpu.async_copy(kv_hbm.at[idx_chunk, pl.ds(0, NOPE_DIM)],
                              nope_v, gsem)
        gt = pltpu.async_copy(tail_hbm.at[idx_chunk], tail_v, gsem)
        gt.wait()
        gn.wait()
        row0 = base + j * _CHUNK
        pltpu.sync_copy(nope_v, nope_hbm.at[pl.ds(row0, _CHUNK)])
        pltpu.sync_copy(tail_v, rope_hbm.at[pl.ds(row0, _CHUNK)])


@jax.jit
def _mla_gather(kv2d, kv_tail, loc32):
    mesh = plsc.VectorSubcoreMesh(core_axis_name="c", subcore_axis_name="s")
    gather = functools.partial(
        pl.kernel,
        mesh=mesh,
        out_type=(
            jax.ShapeDtypeStruct((N_TOK, NOPE_DIM), jnp.float32),
            jax.ShapeDtypeStruct((N_TOK, 128), jnp.float32),
        ),
        scratch_types=[
            pltpu.VMEM((_B_PER_W,), jnp.int32),
            pltpu.VMEM((_CHUNK, NOPE_DIM), jnp.float32),
            pltpu.VMEM((_CHUNK, 128), jnp.float32),
            pltpu.SemaphoreType.DMA,
        ],
    )(_gather_body)
    return gather(kv2d, kv_tail, loc32)


def kernel(kv_buffer, loc, cache_k_nope, cache_k_rope):
    kv2d = kv_buffer.reshape(POOL_SIZE, ROW_DIM)
    kv_tail = kv2d[:, ROW_DIM - 128:]
    loc32 = loc.astype(jnp.int32)
    nope, tail = _mla_gather(kv2d, kv_tail, loc32)
    rope = tail[:, 128 - ROPE_DIM:]
    return (nope.reshape(N_TOK, 1, NOPE_DIM).astype(cache_k_nope.dtype),
            rope.reshape(N_TOK, 1, ROPE_DIM).astype(cache_k_rope.dtype))


# speedup vs baseline: 1.6391x; 1.1068x over previous
import functools

import jax
import jax.numpy as jnp
from jax import lax
from jax.experimental import pallas as pl
from jax.experimental.pallas import tpu as pltpu
from jax.experimental.pallas import tpu_sc as plsc

POOL_SIZE = 65536
N_TOK = 16384
NOPE_DIM = 512
ROPE_DIM = 64
ROW_DIM = NOPE_DIM + ROPE_DIM

_NC, _NS = 2, 16
_NW = _NC * _NS
_B_PER_W = N_TOK // _NW
_CHUNK = 64
_N_CHUNKS = _B_PER_W // _CHUNK
_NBUF = 3


def _gather_body(kv_hbm, tail_hbm, loc_hbm, nope_hbm, rope_hbm,
                 idx_v, oidx_v, n0, n1, n2, t0, t1, t2, gsem, wsem):
    wid = lax.axis_index("s") * _NC + lax.axis_index("c")
    base = wid * _B_PER_W
    pltpu.sync_copy(loc_hbm.at[pl.ds(base, _B_PER_W)], idx_v)
    nbufs = (n0, n1, n2)
    tbufs = (t0, t1, t2)

    lane = lax.iota(jnp.int32, 16)
    for j in range(_N_CHUNKS):
        for k in range(4):
            for v in range(_CHUNK // 16):
                t0_ = base + j * _CHUNK + v * 16
                oidx_v[j, k, pl.ds(v * 16, 16)] = lane * 4 + (4 * t0_ + k)

    def start_gathers(j):
        idx_chunk = idx_v.at[pl.ds(j * _CHUNK, _CHUNK)]
        gn = pltpu.async_copy(kv_hbm.at[idx_chunk, pl.ds(0, NOPE_DIM)],
                              nbufs[j % _NBUF], gsem)
        gt = pltpu.async_copy(tail_hbm.at[idx_chunk], tbufs[j % _NBUF], gsem)
        return gn, gt

    def start_writes(j):
        nv = nbufs[j % _NBUF]
        tv = tbufs[j % _NBUF]
        ws = []
        for k in range(4):
            ws.append(pltpu.async_copy(nv.at[:, pl.ds(128 * k, 128)],
                                       nope_hbm.at[oidx_v.at[j, k]], wsem))
        row0 = base + j * _CHUNK
        ws.append(pltpu.async_copy(tv, rope_hbm.at[pl.ds(row0, _CHUNK)], wsem))
        return ws

    g = {0: start_gathers(0)}
    w = {}
    for j in range(_N_CHUNKS):
        if j + 1 < _N_CHUNKS:
            if j - 2 >= 0:
                for c in w.pop(j - 2):
                    c.wait()
            g[j + 1] = start_gathers(j + 1)
        for c in g.pop(j):
            c.wait()
        w[j] = start_writes(j)
    for j in sorted(w):
        for c in w.pop(j):
            c.wait()


@jax.jit
def _mla_gather(kv2d, kv_tail, loc32):
    mesh = plsc.VectorSubcoreMesh(core_axis_name="c", subcore_axis_name="s")
    gather = functools.partial(
        pl.kernel,
        mesh=mesh,
        out_type=(
            jax.ShapeDtypeStruct((4 * N_TOK, 128), jnp.float32),
            jax.ShapeDtypeStruct((N_TOK, 128), jnp.float32),
        ),
        scratch_types=[
            pltpu.VMEM((_B_PER_W,), jnp.int32),
            pltpu.VMEM((_N_CHUNKS, 4, _CHUNK), jnp.int32),
            pltpu.VMEM((_CHUNK, NOPE_DIM), jnp.float32),
            pltpu.VMEM((_CHUNK, NOPE_DIM), jnp.float32),
            pltpu.VMEM((_CHUNK, NOPE_DIM), jnp.float32),
            pltpu.VMEM((_CHUNK, 128), jnp.float32),
            pltpu.VMEM((_CHUNK, 128), jnp.float32),
            pltpu.VMEM((_CHUNK, 128), jnp.float32),
            pltpu.SemaphoreType.DMA,
            pltpu.SemaphoreType.DMA,
        ],
    )(_gather_body)
    return gather(kv2d, kv_tail, loc32)


def kernel(kv_buffer, loc, cache_k_nope, cache_k_rope):
    kv2d = kv_buffer.reshape(POOL_SIZE, ROW_DIM)
    kv_tail = kv2d[:, ROW_DIM - 128:]
    loc32 = loc.astype(jnp.int32)
    nope4, tail = _mla_gather(kv2d, kv_tail, loc32)
    nope = nope4.reshape(N_TOK, NOPE_DIM)
    rope = tail[:, 128 - ROPE_DIM:]
    return (nope.reshape(N_TOK, 1, NOPE_DIM).astype(cache_k_nope.dtype),
            rope.reshape(N_TOK, 1, ROPE_DIM).astype(cache_k_rope.dtype))
